# baseline (speedup 1.0000x reference)
import jax
import jax.numpy as jnp
from jax.experimental import pallas as pl

_BLK = 256
_EBLK = 512
_NGRAPH = 128


def _pad_rows(a, blk):
    p = (-a.shape[0]) % blk
    if p:
        a = jnp.pad(a, ((0, p), (0, 0)))
    return a


def _lin_kernel(x_ref, w_ref, b_ref, o_ref):
    o_ref[...] = x_ref[...] @ w_ref[...] + b_ref[...]


def _linear(x, W, b):
    n = x.shape[0]
    xp = _pad_rows(x, _BLK)
    npad = xp.shape[0]
    din, dout = W.shape
    out = pl.pallas_call(
        _lin_kernel,
        grid=(npad // _BLK,),
        in_specs=[
            pl.BlockSpec((_BLK, din), lambda i: (i, 0)),
            pl.BlockSpec((din, dout), lambda i: (0, 0)),
            pl.BlockSpec((1, dout), lambda i: (0, 0)),
        ],
        out_specs=pl.BlockSpec((_BLK, dout), lambda i: (i, 0)),
        out_shape=jax.ShapeDtypeStruct((npad, dout), x.dtype),
    )(xp, W, b.reshape(1, dout))
    return out[:n]


def _gin_mlp_kernel(seg_ref, h_ref, w1_ref, b1_ref, w2_ref, b2_ref, o_ref):
    t = seg_ref[...] + 2.0 * h_ref[...]
    t = jnp.maximum(t @ w1_ref[...] + b1_ref[...], 0.0)
    o_ref[...] = jnp.maximum(t @ w2_ref[...] + b2_ref[...], 0.0)


def _gin_mlp(seg, h, p):
    n = h.shape[0]
    segp = _pad_rows(seg, _BLK)
    hp = _pad_rows(h, _BLK)
    npad = hp.shape[0]
    d = h.shape[1]
    out = pl.pallas_call(
        _gin_mlp_kernel,
        grid=(npad // _BLK,),
        in_specs=[
            pl.BlockSpec((_BLK, d), lambda i: (i, 0)),
            pl.BlockSpec((_BLK, d), lambda i: (i, 0)),
            pl.BlockSpec((d, d), lambda i: (0, 0)),
            pl.BlockSpec((1, d), lambda i: (0, 0)),
            pl.BlockSpec((d, d), lambda i: (0, 0)),
            pl.BlockSpec((1, d), lambda i: (0, 0)),
        ],
        out_specs=pl.BlockSpec((_BLK, d), lambda i: (i, 0)),
        out_shape=jax.ShapeDtypeStruct((npad, d), h.dtype),
    )(segp, hp, p["l1"]["W"], p["l1"]["b"].reshape(1, d),
      p["l2"]["W"], p["l2"]["b"].reshape(1, d))
    return out[:n]


def _edge_dot_kernel(a_ref, b_ref, o_ref):
    o_ref[...] = jnp.sum(a_ref[...] * b_ref[...], axis=1, keepdims=True)


def _edge_dot(a, b):
    e, d = a.shape
    out = pl.pallas_call(
        _edge_dot_kernel,
        grid=(e // _EBLK,),
        in_specs=[
            pl.BlockSpec((_EBLK, d), lambda i: (i, 0)),
            pl.BlockSpec((_EBLK, d), lambda i: (i, 0)),
        ],
        out_specs=pl.BlockSpec((_EBLK, 1), lambda i: (i, 0)),
        out_shape=jax.ShapeDtypeStruct((e, 1), a.dtype),
    )(a, b)
    return out[:, 0]


def _readout_kernel(s_ref, wp_ref, bp_ref, wr_ref, br_ref, o_ref):
    t = jnp.maximum(s_ref[...] @ wp_ref[...] + bp_ref[...], 0.0)
    y = t @ wr_ref[...] + br_ref[...]
    m = jnp.max(y, axis=1, keepdims=True)
    o_ref[...] = y - (m + jnp.log(jnp.sum(jnp.exp(y - m), axis=1, keepdims=True)))


def _readout(s, post, read):
    g, d = s.shape
    nc = read["W"].shape[1]
    return pl.pallas_call(
        _readout_kernel,
        in_specs=[
            pl.BlockSpec((g, d), lambda: (0, 0)),
            pl.BlockSpec((d, d), lambda: (0, 0)),
            pl.BlockSpec((1, d), lambda: (0, 0)),
            pl.BlockSpec((d, nc), lambda: (0, 0)),
            pl.BlockSpec((1, nc), lambda: (0, 0)),
        ],
        out_specs=pl.BlockSpec((g, nc), lambda: (0, 0)),
        out_shape=jax.ShapeDtypeStruct((g, nc), s.dtype),
    )(s, post["W"], post["b"].reshape(1, d), read["W"], read["b"].reshape(1, nc))


def kernel(x, edge_index, batch, params):
    n = x.shape[0]
    e = edge_index.shape[1]

    h = _linear(x, params["pre"]["W"], params["pre"]["b"])

    src, dst = edge_index[0], edge_index[1]
    seg = jax.ops.segment_sum(h[src], dst, num_segments=n)
    inv_x = _gin_mlp(seg, h, params["c3"][-1])

    edge_weight = _edge_dot(inv_x[src], inv_x[dst])

    k = e // 2
    _, top_idx = jax.lax.top_k(edge_weight, k)
    mask = jnp.zeros((e,), dtype=bool).at[top_idx].set(True)
    top_pos = jnp.nonzero(mask, size=k)[0]
    rem_pos = jnp.nonzero(~mask, size=e - k)[0]
    top_ei = edge_index[:, top_pos]
    top_ew = edge_weight[top_pos]
    rem_ei = edge_index[:, rem_pos]
    rem_ew = edge_weight[rem_pos]

    seg1 = jax.ops.segment_sum(top_ew[:, None] * h[top_ei[0]], top_ei[1],
                               num_segments=n)
    x1 = _gin_mlp(seg1, h, params["c1"][-1])

    seg2 = jax.ops.segment_sum(rem_ew[:, None] * h[rem_ei[0]], rem_ei[1],
                               num_segments=n)
    x2 = _gin_mlp(seg2, h, params["c4"][-1])

    s = jax.ops.segment_sum(x1 + x2, batch, num_segments=_NGRAPH)
    x5 = _readout(s, params["post"], params["read"])

    return (x5, h, x1, rem_ei, rem_ew, batch)


# speedup vs baseline: 1.0606x; 1.0606x over previous
import jax
import jax.numpy as jnp
from jax.experimental import pallas as pl

_BLK = 256
_EBLK = 512
_NGRAPH = 128


def _pad_rows(a, blk):
    p = (-a.shape[0]) % blk
    if p:
        a = jnp.pad(a, ((0, p), (0, 0)))
    return a


def _lin_kernel(x_ref, w_ref, b_ref, o_ref):
    o_ref[...] = x_ref[...] @ w_ref[...] + b_ref[...]


def _linear(x, W, b):
    n = x.shape[0]
    xp = _pad_rows(x, _BLK)
    npad = xp.shape[0]
    din, dout = W.shape
    out = pl.pallas_call(
        _lin_kernel,
        grid=(npad // _BLK,),
        in_specs=[
            pl.BlockSpec((_BLK, din), lambda i: (i, 0)),
            pl.BlockSpec((din, dout), lambda i: (0, 0)),
            pl.BlockSpec((1, dout), lambda i: (0, 0)),
        ],
        out_specs=pl.BlockSpec((_BLK, dout), lambda i: (i, 0)),
        out_shape=jax.ShapeDtypeStruct((npad, dout), x.dtype),
    )(xp, W, b.reshape(1, dout))
    return out[:n]


def _gin_mlp_kernel(seg_ref, h_ref, w1_ref, b1_ref, w2_ref, b2_ref, o_ref):
    t = seg_ref[...] + 2.0 * h_ref[...]
    t = jnp.maximum(t @ w1_ref[...] + b1_ref[...], 0.0)
    o_ref[...] = jnp.maximum(t @ w2_ref[...] + b2_ref[...], 0.0)


def _gin_mlp(seg, h, p):
    n = h.shape[0]
    segp = _pad_rows(seg, _BLK)
    hp = _pad_rows(h, _BLK)
    npad = hp.shape[0]
    d = h.shape[1]
    out = pl.pallas_call(
        _gin_mlp_kernel,
        grid=(npad // _BLK,),
        in_specs=[
            pl.BlockSpec((_BLK, d), lambda i: (i, 0)),
            pl.BlockSpec((_BLK, d), lambda i: (i, 0)),
            pl.BlockSpec((d, d), lambda i: (0, 0)),
            pl.BlockSpec((1, d), lambda i: (0, 0)),
            pl.BlockSpec((d, d), lambda i: (0, 0)),
            pl.BlockSpec((1, d), lambda i: (0, 0)),
        ],
        out_specs=pl.BlockSpec((_BLK, d), lambda i: (i, 0)),
        out_shape=jax.ShapeDtypeStruct((npad, d), h.dtype),
    )(segp, hp, p["l1"]["W"], p["l1"]["b"].reshape(1, d),
      p["l2"]["W"], p["l2"]["b"].reshape(1, d))
    return out[:n]


def _edge_dot_kernel(a_ref, b_ref, o_ref):
    o_ref[...] = jnp.sum(a_ref[...] * b_ref[...], axis=1, keepdims=True)


def _edge_dot(a, b):
    e, d = a.shape
    out = pl.pallas_call(
        _edge_dot_kernel,
        grid=(e // _EBLK,),
        in_specs=[
            pl.BlockSpec((_EBLK, d), lambda i: (i, 0)),
            pl.BlockSpec((_EBLK, d), lambda i: (i, 0)),
        ],
        out_specs=pl.BlockSpec((_EBLK, 1), lambda i: (i, 0)),
        out_shape=jax.ShapeDtypeStruct((e, 1), a.dtype),
    )(a, b)
    return out[:, 0]


def _readout_kernel(s_ref, wp_ref, bp_ref, wr_ref, br_ref, o_ref):
    t = jnp.maximum(s_ref[...] @ wp_ref[...] + bp_ref[...], 0.0)
    y = t @ wr_ref[...] + br_ref[...]
    m = jnp.max(y, axis=1, keepdims=True)
    o_ref[...] = y - (m + jnp.log(jnp.sum(jnp.exp(y - m), axis=1, keepdims=True)))


def _readout(s, post, read):
    g, d = s.shape
    nc = read["W"].shape[1]
    return pl.pallas_call(
        _readout_kernel,
        in_specs=[
            pl.BlockSpec((g, d), lambda: (0, 0)),
            pl.BlockSpec((d, d), lambda: (0, 0)),
            pl.BlockSpec((1, d), lambda: (0, 0)),
            pl.BlockSpec((d, nc), lambda: (0, 0)),
            pl.BlockSpec((1, nc), lambda: (0, 0)),
        ],
        out_specs=pl.BlockSpec((g, nc), lambda: (0, 0)),
        out_shape=jax.ShapeDtypeStruct((g, nc), s.dtype),
    )(s, post["W"], post["b"].reshape(1, d), read["W"], read["b"].reshape(1, nc))


def kernel(x, edge_index, batch, params):
    n = x.shape[0]
    e = edge_index.shape[1]

    h = _linear(x, params["pre"]["W"], params["pre"]["b"])

    src, dst = edge_index[0], edge_index[1]
    seg = jax.ops.segment_sum(h[src], dst, num_segments=n)
    inv_x = _gin_mlp(seg, h, params["c3"][-1])

    edge_weight = jnp.sum(inv_x[src] * inv_x[dst], axis=1)

    k = e // 2
    _, top_idx = jax.lax.top_k(edge_weight, k)
    mask = jnp.zeros((e,), dtype=bool).at[top_idx].set(True)
    top_pos = jnp.nonzero(mask, size=k)[0]
    rem_pos = jnp.nonzero(~mask, size=e - k)[0]
    top_ei = edge_index[:, top_pos]
    top_ew = edge_weight[top_pos]
    rem_ei = edge_index[:, rem_pos]
    rem_ew = edge_weight[rem_pos]

    seg1 = jax.ops.segment_sum(top_ew[:, None] * h[top_ei[0]], top_ei[1],
                               num_segments=n)
    x1 = _gin_mlp(seg1, h, params["c1"][-1])

    seg2 = jax.ops.segment_sum(rem_ew[:, None] * h[rem_ei[0]], rem_ei[1],
                               num_segments=n)
    x2 = _gin_mlp(seg2, h, params["c4"][-1])

    s = jax.ops.segment_sum(x1 + x2, batch, num_segments=_NGRAPH)
    x5 = _readout(s, params["post"], params["read"])

    return (x5, h, x1, rem_ei, rem_ew, batch)
